# baseline (speedup 1.0000x reference)
import jax
import jax.numpy as jnp
from jax import lax
from jax.experimental import pallas as pl
from jax.experimental.pallas import tpu as pltpu
from jax.experimental.pallas import tpu_sc as plsc

ROWS = 4096
D_IN = 4096

NC = 2
NS = 16
BLK = 128
NBLK_PER_SC = D_IN // NC // BLK
BLK_FLOATS = BLK * ROWS
TILE_SLICE = BLK_FLOATS // NS
ZCHUNK = 16384
ROWLEN = 128


def _build_scatter(ch_per_tile):
    nrows = ch_per_tile // ROWLEN

    def body(child_hbm, parent_hbm, val_hbm, wt_hbm,
             gflat_v, parent_v, val_v, fidx_v, mval_v, zeros_v, flush_v,
             shared_v, sem):
        c = lax.axis_index("c")
        s = lax.axis_index("s")
        pltpu.sync_copy(child_hbm.at[s], gflat_v)
        pltpu.sync_copy(parent_hbm.at[s], parent_v)
        pltpu.sync_copy(val_hbm.at[s], val_v)

        def zinit(i, carry):
            zeros_v[pl.ds(i * 16, 16)] = jnp.zeros((16,), jnp.float32)
            return carry
        lax.fori_loop(0, ZCHUNK // 16, zinit, 0)

        def ginit(j, carry):
            for u in range(ROWLEN // 16):
                col = pl.ds(u * 16, 16)
                gflat_v[j, col] = parent_v[j, col] * ROWS + gflat_v[j, col]
            return carry
        lax.fori_loop(0, nrows, ginit, 0)

        for b in range(NBLK_PER_SC):
            gblk = c * NBLK_PER_SC + b
            lo = gblk * BLK_FLOATS
            for z in range(TILE_SLICE // ZCHUNK):
                pltpu.sync_copy(
                    zeros_v,
                    shared_v.at[pl.ds(s * TILE_SLICE + z * ZCHUNK, ZCHUNK)])
            plsc.subcore_barrier()

            def row_loop(j, carry):
                for u in range(ROWLEN // 16):
                    col = pl.ds(u * 16, 16)
                    rel = gflat_v[j, col] - lo
                    inb = (rel >= 0) & (rel < BLK_FLOATS)
                    fidx_v[j, col] = jnp.where(inb, rel, 0)
                    mval_v[j, col] = jnp.where(inb, val_v[j, col], 0.0)
                pltpu.async_copy(mval_v.at[j], shared_v.at[fidx_v.at[j]],
                                 sem, add=True)
                return carry
            lax.fori_loop(0, nrows, row_loop, 0)
            pltpu.make_async_copy(val_hbm.at[s], mval_v, sem).wait()
            pltpu.sync_copy(shared_v.at[fidx_v.at[nrows - 1]], flush_v)
            pltpu.sync_copy(shared_v.at[fidx_v.at[nrows - 2]], flush_v)
            pltpu.sync_copy(shared_v.at[fidx_v.at[nrows - 3]], flush_v)
            plsc.subcore_barrier()
            pltpu.sync_copy(
                shared_v.at[pl.ds(s * TILE_SLICE, TILE_SLICE)],
                wt_hbm.at[pl.ds(gblk * BLK_FLOATS + s * TILE_SLICE,
                                TILE_SLICE)])
            plsc.subcore_barrier()

    return pl.kernel(
        body,
        out_type=jax.ShapeDtypeStruct((D_IN * ROWS,), jnp.float32),
        mesh=plsc.VectorSubcoreMesh(core_axis_name="c", subcore_axis_name="s"),
        scratch_types=[
            pltpu.VMEM((nrows, ROWLEN), jnp.int32),
            pltpu.VMEM((nrows, ROWLEN), jnp.int32),
            pltpu.VMEM((nrows, ROWLEN), jnp.float32),
            pltpu.VMEM((nrows, ROWLEN), jnp.int32),
            pltpu.VMEM((nrows, ROWLEN), jnp.float32),
            pltpu.VMEM((ZCHUNK,), jnp.float32),
            pltpu.VMEM((ROWLEN,), jnp.float32),
            pltpu.VMEM_SHARED((BLK_FLOATS,), jnp.float32),
            pltpu.SemaphoreType.DMA,
        ],
    )


def _mm_body(x_ref, w_ref, o_ref):
    o_ref[...] = jnp.maximum(
        jnp.dot(x_ref[...], w_ref[...], preferred_element_type=jnp.float32),
        0.0)


def _matmul_relu(x, wt):
    batch = x.shape[0]
    nbn = 8
    bn = ROWS // nbn
    return pl.pallas_call(
        _mm_body,
        grid=(nbn,),
        in_specs=[
            pl.BlockSpec((batch, D_IN), lambda j: (0, 0)),
            pl.BlockSpec((D_IN, bn), lambda j: (0, j)),
        ],
        out_specs=pl.BlockSpec((batch, bn), lambda j: (0, j)),
        out_shape=jax.ShapeDtypeStruct((batch, ROWS), jnp.float32),
    )(x, wt)


def kernel(input, weight_indices, weight_values):
    child = weight_indices[0].astype(jnp.int32)
    parent = weight_indices[1].astype(jnp.int32)
    vals = weight_values.astype(jnp.float32)
    nnz = vals.shape[0]

    ch_per_tile = -(-nnz // NS)
    ch_per_tile = -(-ch_per_tile // ROWLEN) * ROWLEN
    pad = ch_per_tile * NS - nnz
    nrows = ch_per_tile // ROWLEN
    shape3 = (NS, nrows, ROWLEN)
    child = jnp.concatenate([child, jnp.zeros((pad,), jnp.int32)]).reshape(shape3)
    parent = jnp.concatenate([parent, jnp.zeros((pad,), jnp.int32)]).reshape(shape3)
    vals = jnp.concatenate([vals, jnp.zeros((pad,), jnp.float32)]).reshape(shape3)

    wt_flat = _build_scatter(ch_per_tile)(child, parent, vals)
    wt = wt_flat.reshape(D_IN, ROWS)
    out = _matmul_relu(input, wt)
    return jnp.concatenate([input, out], axis=1)


# speedup vs baseline: 10.8794x; 10.8794x over previous
import jax
import jax.numpy as jnp
from jax import lax
from jax.experimental import pallas as pl
from jax.experimental.pallas import tpu as pltpu
from jax.experimental.pallas import tpu_sc as plsc

ROWS = 4096
D_IN = 4096

NC = 2
NS = 16
BLK = 128
NBLK_PER_SC = D_IN // NC // BLK
BLK_FLOATS = BLK * ROWS
TILE_SLICE = BLK_FLOATS // NS
ZCHUNK = 16384
DUMP = 2048
ROWLEN = 128


def _build_scatter(ch_per_tile):
    nrows = ch_per_tile // ROWLEN

    def body(child_hbm, parent_hbm, val_hbm, wt_hbm,
             gflat_v, parent_v, val_v, fidx_v, mval_v,
             zeros_v, flush_v, shared_v, sem):
        c = lax.axis_index("c")
        s = lax.axis_index("s")
        pltpu.sync_copy(child_hbm.at[s], gflat_v)
        pltpu.sync_copy(parent_hbm.at[s], parent_v)
        pltpu.sync_copy(val_hbm.at[s], val_v)

        def zinit(i, carry):
            zeros_v[pl.ds(i * 16, 16)] = jnp.zeros((16,), jnp.float32)
            return carry
        lax.fori_loop(0, ZCHUNK // 16, zinit, 0)

        def ginit(j, carry):
            for u in range(ROWLEN // 16):
                col = pl.ds(u * 16, 16)
                gflat_v[j, col] = parent_v[j, col] * ROWS + gflat_v[j, col]
            return carry
        lax.fori_loop(0, nrows, ginit, 0)

        for b in range(NBLK_PER_SC):
            gblk = c * NBLK_PER_SC + b
            lo = gblk * BLK_FLOATS
            for z in range(TILE_SLICE // ZCHUNK):
                pltpu.sync_copy(
                    zeros_v,
                    shared_v.at[pl.ds(s * TILE_SLICE + z * ZCHUNK, ZCHUNK)])
            plsc.subcore_barrier()

            def row_loop(j, carry):
                dbase = BLK_FLOATS + ((j * ROWLEN) & (DUMP - 1))
                for u in range(ROWLEN // 16):
                    col = pl.ds(u * 16, 16)
                    rel = gflat_v[j, col] - lo
                    inb = (rel >= 0) & (rel < BLK_FLOATS)
                    dump = dbase + u * 16 + lax.iota(jnp.int32, 16)
                    fidx_v[j, col] = jnp.where(inb, rel, dump)
                    mval_v[j, col] = jnp.where(inb, val_v[j, col], 0.0)
                pltpu.async_copy(mval_v.at[j], shared_v.at[fidx_v.at[j]],
                                 sem, add=True)
                return carry
            lax.fori_loop(0, nrows, row_loop, 0)
            pltpu.make_async_copy(val_hbm.at[s], mval_v, sem).wait()
            pltpu.sync_copy(shared_v.at[fidx_v.at[nrows - 1]], flush_v)
            pltpu.sync_copy(shared_v.at[fidx_v.at[nrows - 2]], flush_v)
            pltpu.sync_copy(shared_v.at[fidx_v.at[nrows - 3]], flush_v)
            plsc.subcore_barrier()
            pltpu.sync_copy(
                shared_v.at[pl.ds(s * TILE_SLICE, TILE_SLICE)],
                wt_hbm.at[pl.ds(gblk * BLK_FLOATS + s * TILE_SLICE,
                                TILE_SLICE)])
            plsc.subcore_barrier()

    return pl.kernel(
        body,
        out_type=jax.ShapeDtypeStruct((D_IN * ROWS,), jnp.float32),
        mesh=plsc.VectorSubcoreMesh(core_axis_name="c", subcore_axis_name="s"),
        scratch_types=[
            pltpu.VMEM((nrows, ROWLEN), jnp.int32),
            pltpu.VMEM((nrows, ROWLEN), jnp.int32),
            pltpu.VMEM((nrows, ROWLEN), jnp.float32),
            pltpu.VMEM((nrows, ROWLEN), jnp.int32),
            pltpu.VMEM((nrows, ROWLEN), jnp.float32),
            pltpu.VMEM((ZCHUNK,), jnp.float32),
            pltpu.VMEM((ROWLEN,), jnp.float32),
            pltpu.VMEM_SHARED((BLK_FLOATS + DUMP,), jnp.float32),
            pltpu.SemaphoreType.DMA,
        ],
    )


def _mm_body(x_ref, w_ref, o_ref):
    o_ref[...] = jnp.maximum(
        jnp.dot(x_ref[...], w_ref[...], preferred_element_type=jnp.float32),
        0.0)


def _matmul_relu(x, wt):
    batch = x.shape[0]
    nbn = 8
    bn = ROWS // nbn
    return pl.pallas_call(
        _mm_body,
        grid=(nbn,),
        in_specs=[
            pl.BlockSpec((batch, D_IN), lambda j: (0, 0)),
            pl.BlockSpec((D_IN, bn), lambda j: (0, j)),
        ],
        out_specs=pl.BlockSpec((batch, bn), lambda j: (0, j)),
        out_shape=jax.ShapeDtypeStruct((batch, ROWS), jnp.float32),
    )(x, wt)


def kernel(input, weight_indices, weight_values):
    child = weight_indices[0].astype(jnp.int32)
    parent = weight_indices[1].astype(jnp.int32)
    vals = weight_values.astype(jnp.float32)
    nnz = vals.shape[0]

    ch_per_tile = -(-nnz // NS)
    ch_per_tile = -(-ch_per_tile // ROWLEN) * ROWLEN
    pad = ch_per_tile * NS - nnz
    nrows = ch_per_tile // ROWLEN
    shape3 = (NS, nrows, ROWLEN)
    child = jnp.concatenate([child, jnp.zeros((pad,), jnp.int32)]).reshape(shape3)
    parent = jnp.concatenate([parent, jnp.zeros((pad,), jnp.int32)]).reshape(shape3)
    vals = jnp.concatenate([vals, jnp.zeros((pad,), jnp.float32)]).reshape(shape3)

    wt_flat = _build_scatter(ch_per_tile)(child, parent, vals)
    wt = wt_flat.reshape(D_IN, ROWS)
    out = _matmul_relu(input, wt)
    return jnp.concatenate([input, out], axis=1)


# speedup vs baseline: 12.2073x; 1.1221x over previous
import jax
import jax.numpy as jnp
from jax import lax
from jax.experimental import pallas as pl
from jax.experimental.pallas import tpu as pltpu
from jax.experimental.pallas import tpu_sc as plsc

ROWS = 4096
D_IN = 4096

NC = 2
NS = 16
BLK = 256
NBLK_PER_SC = D_IN // NC // BLK
BLK_FLOATS = BLK * ROWS
TILE_SLICE = BLK_FLOATS // NS
ZCHUNK = 16384
DUMP = 2048
ROWLEN = 128


def _build_scatter(ch_per_tile):
    nrows = ch_per_tile // ROWLEN

    def body(gflat_hbm, val_hbm, wt_hbm,
             gflat_v, val_v, fidx_v, mval_v,
             zeros_v, flush_v, shared_v, sem):
        c = lax.axis_index("c")
        s = lax.axis_index("s")
        pltpu.sync_copy(gflat_hbm.at[s], gflat_v)
        pltpu.sync_copy(val_hbm.at[s], val_v)

        def zinit(i, carry):
            zeros_v[pl.ds(i * 16, 16)] = jnp.zeros((16,), jnp.float32)
            return carry
        lax.fori_loop(0, ZCHUNK // 16, zinit, 0)

        for b in range(NBLK_PER_SC):
            gblk = c * NBLK_PER_SC + b
            lo = gblk * BLK_FLOATS
            for z in range(TILE_SLICE // ZCHUNK):
                pltpu.sync_copy(
                    zeros_v,
                    shared_v.at[pl.ds(s * TILE_SLICE + z * ZCHUNK, ZCHUNK)])
            plsc.subcore_barrier()

            def row_loop(j, carry):
                dbase = BLK_FLOATS + ((j * ROWLEN) & (DUMP - 1))
                for u in range(ROWLEN // 16):
                    col = pl.ds(u * 16, 16)
                    rel = gflat_v[j, col] - lo
                    inb = (rel >= 0) & (rel < BLK_FLOATS)
                    dump = dbase + u * 16 + lax.iota(jnp.int32, 16)
                    fidx_v[j, col] = jnp.where(inb, rel, dump)
                    mval_v[j, col] = jnp.where(inb, val_v[j, col], 0.0)
                pltpu.async_copy(mval_v.at[j], shared_v.at[fidx_v.at[j]],
                                 sem, add=True)
                return carry
            lax.fori_loop(0, nrows, row_loop, 0)
            pltpu.make_async_copy(val_hbm.at[s], mval_v, sem).wait()
            pltpu.sync_copy(shared_v.at[fidx_v.at[nrows - 1]], flush_v)
            pltpu.sync_copy(shared_v.at[fidx_v.at[nrows - 2]], flush_v)
            pltpu.sync_copy(shared_v.at[fidx_v.at[nrows - 3]], flush_v)
            plsc.subcore_barrier()
            pltpu.sync_copy(
                shared_v.at[pl.ds(s * TILE_SLICE, TILE_SLICE)],
                wt_hbm.at[pl.ds(gblk * BLK_FLOATS + s * TILE_SLICE,
                                TILE_SLICE)])
            plsc.subcore_barrier()

    return pl.kernel(
        body,
        out_type=jax.ShapeDtypeStruct((D_IN * ROWS,), jnp.float32),
        mesh=plsc.VectorSubcoreMesh(core_axis_name="c", subcore_axis_name="s"),
        scratch_types=[
            pltpu.VMEM((nrows, ROWLEN), jnp.int32),
            pltpu.VMEM((nrows, ROWLEN), jnp.float32),
            pltpu.VMEM((nrows, ROWLEN), jnp.int32),
            pltpu.VMEM((nrows, ROWLEN), jnp.float32),
            pltpu.VMEM((ZCHUNK,), jnp.float32),
            pltpu.VMEM((ROWLEN,), jnp.float32),
            pltpu.VMEM_SHARED((BLK_FLOATS + DUMP,), jnp.float32),
            pltpu.SemaphoreType.DMA,
        ],
    )


def _mm_body(x_ref, w_ref, o_ref):
    o_ref[...] = jnp.maximum(
        jnp.dot(x_ref[...], w_ref[...], preferred_element_type=jnp.float32),
        0.0)


def _matmul_relu(x, wt):
    batch = x.shape[0]
    nbn = 8
    bn = ROWS // nbn
    return pl.pallas_call(
        _mm_body,
        grid=(nbn,),
        in_specs=[
            pl.BlockSpec((batch, D_IN), lambda j: (0, 0)),
            pl.BlockSpec((D_IN, bn), lambda j: (0, j)),
        ],
        out_specs=pl.BlockSpec((batch, bn), lambda j: (0, j)),
        out_shape=jax.ShapeDtypeStruct((batch, ROWS), jnp.float32),
    )(x, wt)


def kernel(input, weight_indices, weight_values):
    child = weight_indices[0].astype(jnp.int32)
    parent = weight_indices[1].astype(jnp.int32)
    vals = weight_values.astype(jnp.float32)
    nnz = vals.shape[0]

    ch_per_tile = -(-nnz // NS)
    ch_per_tile = -(-ch_per_tile // ROWLEN) * ROWLEN
    pad = ch_per_tile * NS - nnz
    nrows = ch_per_tile // ROWLEN
    shape3 = (NS, nrows, ROWLEN)
    gflat = parent * ROWS + child
    gflat = jnp.concatenate([gflat, jnp.zeros((pad,), jnp.int32)]).reshape(shape3)
    vals = jnp.concatenate([vals, jnp.zeros((pad,), jnp.float32)]).reshape(shape3)

    wt_flat = _build_scatter(ch_per_tile)(gflat, vals)
    wt = wt_flat.reshape(D_IN, ROWS)
    out = _matmul_relu(input, wt)
    return jnp.concatenate([input, out], axis=1)


# speedup vs baseline: 13.0319x; 1.0675x over previous
import jax
import jax.numpy as jnp
from jax import lax
from jax.experimental import pallas as pl
from jax.experimental.pallas import tpu as pltpu
from jax.experimental.pallas import tpu_sc as plsc

ROWS = 4096
D_IN = 4096

NC = 2
NS = 16
BLK = 256
NBLK_PER_SC = D_IN // NC // BLK
BLK_FLOATS = BLK * ROWS
TILE_SLICE = BLK_FLOATS // NS
ZCHUNK = 16384
DUMP = 2048
ROWLEN = 128


def _build_scatter(ch_per_tile):
    nrows = ch_per_tile // ROWLEN

    def body(gflat_hbm, val_hbm, wt_hbm,
             gflat_v, val_v, fidx_v, mval_v,
             zeros_v, flush_v, shared_v, sem):
        c = lax.axis_index("c")
        s = lax.axis_index("s")
        pltpu.sync_copy(gflat_hbm.at[s], gflat_v)
        pltpu.sync_copy(val_hbm.at[s], val_v)

        def zinit(i, carry):
            zeros_v[pl.ds(i * 16, 16)] = jnp.zeros((16,), jnp.float32)
            return carry
        lax.fori_loop(0, ZCHUNK // 16, zinit, 0)

        for b in range(NBLK_PER_SC):
            gblk = c * NBLK_PER_SC + b
            lo = gblk * BLK_FLOATS
            for z in range(TILE_SLICE // ZCHUNK):
                pltpu.sync_copy(
                    zeros_v,
                    shared_v.at[pl.ds(s * TILE_SLICE + z * ZCHUNK, ZCHUNK)])
            plsc.subcore_barrier()

            def row_loop(j, carry):
                dbase = BLK_FLOATS + ((j * ROWLEN) & (DUMP - 1))
                for u in range(ROWLEN // 16):
                    col = pl.ds(u * 16, 16)
                    rel = gflat_v[j, col] - lo
                    inb = (rel >= 0) & (rel < BLK_FLOATS)
                    dump = dbase + u * 16 + lax.iota(jnp.int32, 16)
                    fidx_v[j, col] = jnp.where(inb, rel, dump)
                    mval_v[j, col] = jnp.where(inb, val_v[j, col], 0.0)
                pltpu.async_copy(mval_v.at[j], shared_v.at[fidx_v.at[j]],
                                 sem, add=True)
                return carry
            lax.fori_loop(0, nrows, row_loop, 0)
            pltpu.make_async_copy(val_hbm.at[s], mval_v, sem).wait()
            pltpu.sync_copy(shared_v.at[fidx_v.at[nrows - 1]], flush_v)
            pltpu.sync_copy(shared_v.at[fidx_v.at[nrows - 2]], flush_v)
            pltpu.sync_copy(shared_v.at[fidx_v.at[nrows - 3]], flush_v)
            plsc.subcore_barrier()
            pltpu.sync_copy(
                shared_v.at[pl.ds(s * TILE_SLICE, TILE_SLICE)],
                wt_hbm.at[pl.ds(gblk * BLK_FLOATS + s * TILE_SLICE,
                                TILE_SLICE)])
            plsc.subcore_barrier()

    return pl.kernel(
        body,
        out_type=jax.ShapeDtypeStruct((D_IN * ROWS,), jnp.float32),
        mesh=plsc.VectorSubcoreMesh(core_axis_name="c", subcore_axis_name="s"),
        scratch_types=[
            pltpu.VMEM((nrows, ROWLEN), jnp.int32),
            pltpu.VMEM((nrows, ROWLEN), jnp.float32),
            pltpu.VMEM((nrows, ROWLEN), jnp.int32),
            pltpu.VMEM((nrows, ROWLEN), jnp.float32),
            pltpu.VMEM((ZCHUNK,), jnp.float32),
            pltpu.VMEM((ROWLEN,), jnp.float32),
            pltpu.VMEM_SHARED((BLK_FLOATS + DUMP,), jnp.float32),
            pltpu.SemaphoreType.DMA,
        ],
    )


def _mm_body(x_ref, w_ref, o_ref):
    j = pl.program_id(0)

    @pl.when(j < 8)
    def _copy():
        o_ref[...] = x_ref[:, pl.ds(j * 512, 512)]

    @pl.when(j >= 8)
    def _mm():
        o_ref[...] = jnp.maximum(
            jnp.dot(x_ref[...], w_ref[...],
                    preferred_element_type=jnp.float32),
            0.0)


def _matmul_relu_concat(x, wt):
    batch = x.shape[0]
    bn = 512
    return pl.pallas_call(
        _mm_body,
        grid=(16,),
        in_specs=[
            pl.BlockSpec((batch, D_IN), lambda j: (0, 0)),
            pl.BlockSpec((D_IN, bn), lambda j: (0, jnp.maximum(j - 8, 0))),
        ],
        out_specs=pl.BlockSpec((batch, bn), lambda j: (0, j)),
        out_shape=jax.ShapeDtypeStruct((batch, 2 * ROWS), jnp.float32),
    )(x, wt)


def kernel(input, weight_indices, weight_values):
    child = weight_indices[0].astype(jnp.int32)
    parent = weight_indices[1].astype(jnp.int32)
    vals = weight_values.astype(jnp.float32)
    nnz = vals.shape[0]

    ch_per_tile = -(-nnz // NS)
    ch_per_tile = -(-ch_per_tile // ROWLEN) * ROWLEN
    pad = ch_per_tile * NS - nnz
    nrows = ch_per_tile // ROWLEN
    shape3 = (NS, nrows, ROWLEN)
    gflat = parent * ROWS + child
    gflat = jnp.concatenate([gflat, jnp.zeros((pad,), jnp.int32)]).reshape(shape3)
    vals = jnp.concatenate([vals, jnp.zeros((pad,), jnp.float32)]).reshape(shape3)

    wt_flat = _build_scatter(ch_per_tile)(gflat, vals)
    wt = wt_flat.reshape(D_IN, ROWS)
    return _matmul_relu_concat(input, wt)


# speedup vs baseline: 13.0429x; 1.0008x over previous
import jax
import jax.numpy as jnp
from jax import lax
from jax.experimental import pallas as pl
from jax.experimental.pallas import tpu as pltpu
from jax.experimental.pallas import tpu_sc as plsc

ROWS = 4096
D_IN = 4096

NC = 2
NS = 16
BLK = 256
NBLK_PER_SC = D_IN // NC // BLK
BLK_FLOATS = BLK * ROWS
TILE_SLICE = BLK_FLOATS // NS
ZCHUNK = 16384
DUMP = 2048
ROWLEN = 128


def _build_scatter(ch_per_tile):
    nrows = ch_per_tile // ROWLEN

    def body(gflat_hbm, val_hbm, wt_hbm,
             gflat_v, val_v, fidx_v, mval_v,
             zeros_v, flush_v, shared_v, sem):
        c = lax.axis_index("c")
        s = lax.axis_index("s")
        pltpu.sync_copy(gflat_hbm.at[s], gflat_v)
        pltpu.sync_copy(val_hbm.at[s], val_v)

        def zinit(i, carry):
            zeros_v[pl.ds(i * 16, 16)] = jnp.zeros((16,), jnp.float32)
            return carry
        lax.fori_loop(0, ZCHUNK // 16, zinit, 0)

        for b in range(NBLK_PER_SC):
            gblk = c * NBLK_PER_SC + b
            lo = gblk * BLK_FLOATS
            for z in range(TILE_SLICE // ZCHUNK):
                pltpu.sync_copy(
                    zeros_v,
                    shared_v.at[pl.ds(s * TILE_SLICE + z * ZCHUNK, ZCHUNK)])
            plsc.subcore_barrier()

            def row_loop(j, carry):
                dbase = BLK_FLOATS + ((j * ROWLEN) & (DUMP - 1))
                for u in range(ROWLEN // 16):
                    col = pl.ds(u * 16, 16)
                    rel = gflat_v[j, col] - lo
                    inb = (rel >= 0) & (rel < BLK_FLOATS)
                    dump = dbase + u * 16 + lax.iota(jnp.int32, 16)
                    fidx_v[j, col] = jnp.where(inb, rel, dump)
                    mval_v[j, col] = jnp.where(inb, val_v[j, col], 0.0)
                pltpu.async_copy(mval_v.at[j], shared_v.at[fidx_v.at[j]],
                                 sem, add=True)
                return carry
            lax.fori_loop(0, nrows, row_loop, 0)
            pltpu.make_async_copy(val_hbm.at[s], mval_v, sem).wait()
            pltpu.sync_copy(shared_v.at[fidx_v.at[nrows - 1]], flush_v)
            pltpu.sync_copy(shared_v.at[fidx_v.at[nrows - 2]], flush_v)
            pltpu.sync_copy(shared_v.at[fidx_v.at[nrows - 3]], flush_v)
            plsc.subcore_barrier()
            pltpu.sync_copy(
                shared_v.at[pl.ds(s * TILE_SLICE, TILE_SLICE)],
                wt_hbm.at[pl.ds(gblk * BLK_FLOATS + s * TILE_SLICE,
                                TILE_SLICE)])
            plsc.subcore_barrier()

    return pl.kernel(
        body,
        out_type=jax.ShapeDtypeStruct((D_IN * ROWS,), jnp.float32),
        mesh=plsc.VectorSubcoreMesh(core_axis_name="c", subcore_axis_name="s"),
        scratch_types=[
            pltpu.VMEM((nrows, ROWLEN), jnp.int32),
            pltpu.VMEM((nrows, ROWLEN), jnp.float32),
            pltpu.VMEM((nrows, ROWLEN), jnp.int32),
            pltpu.VMEM((nrows, ROWLEN), jnp.float32),
            pltpu.VMEM((ZCHUNK,), jnp.float32),
            pltpu.VMEM((ROWLEN,), jnp.float32),
            pltpu.VMEM_SHARED((BLK_FLOATS + DUMP,), jnp.float32),
            pltpu.SemaphoreType.DMA,
        ],
    )


def _mm_body(x_ref, w_ref, o_ref):
    j = pl.program_id(0)

    @pl.when(j < 8)
    def _copy():
        o_ref[...] = x_ref[:, pl.ds(j * 512, 512)]

    @pl.when(j >= 8)
    def _mm():
        o_ref[...] = jnp.maximum(
            jnp.dot(x_ref[...].astype(jnp.bfloat16),
                    w_ref[...].astype(jnp.bfloat16),
                    preferred_element_type=jnp.float32),
            0.0)


def _matmul_relu_concat(x, wt):
    batch = x.shape[0]
    bn = 512
    return pl.pallas_call(
        _mm_body,
        grid=(16,),
        in_specs=[
            pl.BlockSpec((batch, D_IN), lambda j: (0, 0)),
            pl.BlockSpec((D_IN, bn), lambda j: (0, jnp.maximum(j - 8, 0))),
        ],
        out_specs=pl.BlockSpec((batch, bn), lambda j: (0, j)),
        out_shape=jax.ShapeDtypeStruct((batch, 2 * ROWS), jnp.float32),
    )(x, wt)


def kernel(input, weight_indices, weight_values):
    child = weight_indices[0].astype(jnp.int32)
    parent = weight_indices[1].astype(jnp.int32)
    vals = weight_values.astype(jnp.float32)
    nnz = vals.shape[0]

    ch_per_tile = -(-nnz // NS)
    ch_per_tile = -(-ch_per_tile // ROWLEN) * ROWLEN
    pad = ch_per_tile * NS - nnz
    nrows = ch_per_tile // ROWLEN
    shape3 = (NS, nrows, ROWLEN)
    gflat = parent * ROWS + child
    gflat = jnp.concatenate([gflat, jnp.zeros((pad,), jnp.int32)]).reshape(shape3)
    vals = jnp.concatenate([vals, jnp.zeros((pad,), jnp.float32)]).reshape(shape3)

    wt_flat = _build_scatter(ch_per_tile)(gflat, vals)
    wt = wt_flat.reshape(D_IN, ROWS)
    return _matmul_relu_concat(input, wt)
